# Initial kernel scaffold; baseline (speedup 1.0000x reference)
import jax
import jax.numpy as jnp
from jax.experimental import pallas as pl


def kernel(inputs, table, W1, b1, W2, b2):
    raise NotImplementedError("write your pallas kernel here")


# speedup vs baseline: 2.3832x; 2.3832x over previous
import functools

import jax
import jax.numpy as jnp
from jax import lax
from jax.experimental import pallas as pl
from jax.experimental.pallas import tpu as pltpu
from jax.experimental.pallas import tpu_sc as plsc

BATCH = 4096
MAXLEN = 200
EMBED_DIM = 32
HIDDEN = 128
CLASS_NUM = 128

NUM_CORES = 2
NUM_SUBCORES = 16
NUM_WORKERS = NUM_CORES * NUM_SUBCORES
BPW = BATCH // NUM_WORKERS
NBUF = 4
ROUNDS = MAXLEN // NBUF


def _pool_body(idxT_hbm, table_hbm, pool_hbm, idx_v, accs, sems):
    wid = lax.axis_index("s") * NUM_CORES + lax.axis_index("c")
    base = wid * BPW

    pltpu.sync_copy(idxT_hbm.at[:, pl.ds(base, BPW)], idx_v)

    for b in range(NBUF):
        pltpu.async_copy(table_hbm.at[idx_v.at[b]], accs[b], sems[b])

    def round_body(r, carry):
        for b in range(NBUF):
            pltpu.make_async_copy(table_hbm.at[idx_v.at[b]], accs[b], sems[b]).wait()
            pltpu.async_copy(
                table_hbm.at[idx_v.at[r * NBUF + b]], accs[b], sems[b], add=True
            )
        return carry

    lax.fori_loop(1, ROUNDS, round_body, 0)
    for b in range(NBUF):
        pltpu.make_async_copy(table_hbm.at[idx_v.at[b]], accs[b], sems[b]).wait()

    def fold_body(r, carry):
        for c in range(EMBED_DIM // 16):
            sl = pl.ds(c * 16, 16)
            acc = accs[0][r, sl]
            for b in range(1, NBUF):
                acc = acc + accs[b][r, sl]
            accs[0][r, sl] = acc
        return carry

    lax.fori_loop(0, BPW, fold_body, 0)
    pltpu.sync_copy(accs[0], pool_hbm.at[pl.ds(base, BPW)])


def _pool_kernel_body(idxT_hbm, table_hbm, pool_hbm, idx_v, *rest):
    accs = rest[:NBUF]
    sems = rest[NBUF:]
    _pool_body(idxT_hbm, table_hbm, pool_hbm, idx_v, accs, sems)


_pool = pl.kernel(
    _pool_kernel_body,
    out_type=jax.ShapeDtypeStruct((BATCH, EMBED_DIM), jnp.float32),
    mesh=plsc.VectorSubcoreMesh(
        core_axis_name="c",
        subcore_axis_name="s",
        num_cores=NUM_CORES,
        num_subcores=NUM_SUBCORES,
    ),
    scratch_types=(
        [pltpu.VMEM((MAXLEN, BPW), jnp.int32)]
        + [pltpu.VMEM((BPW, EMBED_DIM), jnp.float32) for _ in range(NBUF)]
        + [pltpu.SemaphoreType.DMA for _ in range(NBUF)]
    ),
    compiler_params=pltpu.CompilerParams(use_tc_tiling_on_sc=False),
)


def _mlp_body(pool_ref, w1_ref, b1_ref, w2_ref, b2_ref, out_ref):
    x = pool_ref[...] * (1.0 / MAXLEN)
    h = jnp.dot(x, w1_ref[...], preferred_element_type=jnp.float32) + b1_ref[...]
    h = jnp.maximum(h, 0.0)
    logits = jnp.dot(h, w2_ref[...], preferred_element_type=jnp.float32) + b2_ref[...]
    m = jnp.max(logits, axis=-1, keepdims=True)
    e = jnp.exp(logits - m)
    out_ref[...] = e / jnp.sum(e, axis=-1, keepdims=True)


_mlp = pl.pallas_call(
    _mlp_body,
    out_shape=jax.ShapeDtypeStruct((BATCH, CLASS_NUM), jnp.float32),
)


def kernel(inputs, table, W1, b1, W2, b2):
    idxT = inputs.T
    pool = _pool(idxT, table)
    return _mlp(pool, W1, b1.reshape(1, HIDDEN), W2, b2.reshape(1, CLASS_NUM))


# speedup vs baseline: 2.4364x; 1.0223x over previous
import functools

import jax
import jax.numpy as jnp
from jax import lax
from jax.experimental import pallas as pl
from jax.experimental.pallas import tpu as pltpu
from jax.experimental.pallas import tpu_sc as plsc

BATCH = 4096
MAXLEN = 200
EMBED_DIM = 32
HIDDEN = 128
CLASS_NUM = 128

NUM_CORES = 2
NUM_SUBCORES = 16
NUM_WORKERS = NUM_CORES * NUM_SUBCORES
BPW = BATCH // NUM_WORKERS
NBUF = 8
ROUNDS = MAXLEN // NBUF


def _pool_body(idxT_hbm, table_hbm, pool_hbm, idx_v, accs, sems):
    wid = lax.axis_index("s") * NUM_CORES + lax.axis_index("c")
    base = wid * BPW

    pltpu.sync_copy(idxT_hbm.at[:, pl.ds(base, BPW)], idx_v)

    for b in range(NBUF):
        pltpu.async_copy(table_hbm.at[idx_v.at[b]], accs[b], sems[b])

    def round_body(r, carry):
        for b in range(NBUF):
            pltpu.make_async_copy(table_hbm.at[idx_v.at[b]], accs[b], sems[b]).wait()
            pltpu.async_copy(
                table_hbm.at[idx_v.at[r * NBUF + b]], accs[b], sems[b], add=True
            )
        return carry

    lax.fori_loop(1, ROUNDS, round_body, 0)
    for b in range(NBUF):
        pltpu.make_async_copy(table_hbm.at[idx_v.at[b]], accs[b], sems[b]).wait()

    def fold_body(r, carry):
        for c in range(EMBED_DIM // 16):
            sl = pl.ds(c * 16, 16)
            acc = accs[0][r, sl]
            for b in range(1, NBUF):
                acc = acc + accs[b][r, sl]
            accs[0][r, sl] = acc
        return carry

    lax.fori_loop(0, BPW, fold_body, 0)
    pltpu.sync_copy(accs[0], pool_hbm.at[pl.ds(base, BPW)])


def _pool_kernel_body(idxT_hbm, table_hbm, pool_hbm, idx_v, *rest):
    accs = rest[:NBUF]
    sems = rest[NBUF:]
    _pool_body(idxT_hbm, table_hbm, pool_hbm, idx_v, accs, sems)


_pool = pl.kernel(
    _pool_kernel_body,
    out_type=jax.ShapeDtypeStruct((BATCH, EMBED_DIM), jnp.float32),
    mesh=plsc.VectorSubcoreMesh(
        core_axis_name="c",
        subcore_axis_name="s",
        num_cores=NUM_CORES,
        num_subcores=NUM_SUBCORES,
    ),
    scratch_types=(
        [pltpu.VMEM((MAXLEN, BPW), jnp.int32)]
        + [pltpu.VMEM((BPW, EMBED_DIM), jnp.float32) for _ in range(NBUF)]
        + [pltpu.SemaphoreType.DMA for _ in range(NBUF)]
    ),
    compiler_params=pltpu.CompilerParams(use_tc_tiling_on_sc=False),
)


def _mlp_body(pool_ref, w1_ref, b1_ref, w2_ref, b2_ref, out_ref):
    x = pool_ref[...] * (1.0 / MAXLEN)
    h = jnp.dot(x, w1_ref[...], preferred_element_type=jnp.float32) + b1_ref[...]
    h = jnp.maximum(h, 0.0)
    logits = jnp.dot(h, w2_ref[...], preferred_element_type=jnp.float32) + b2_ref[...]
    m = jnp.max(logits, axis=-1, keepdims=True)
    e = jnp.exp(logits - m)
    out_ref[...] = e / jnp.sum(e, axis=-1, keepdims=True)


_mlp = pl.pallas_call(
    _mlp_body,
    out_shape=jax.ShapeDtypeStruct((BATCH, CLASS_NUM), jnp.float32),
)


def kernel(inputs, table, W1, b1, W2, b2):
    idxT = inputs.T
    pool = _pool(idxT, table)
    return _mlp(pool, W1, b1.reshape(1, HIDDEN), W2, b2.reshape(1, CLASS_NUM))


# speedup vs baseline: 2.4417x; 1.0022x over previous
import functools

import jax
import jax.numpy as jnp
from jax import lax
from jax.experimental import pallas as pl
from jax.experimental.pallas import tpu as pltpu
from jax.experimental.pallas import tpu_sc as plsc

BATCH = 4096
MAXLEN = 200
EMBED_DIM = 32
HIDDEN = 128
CLASS_NUM = 128

NUM_CORES = 2
NUM_SUBCORES = 16
NUM_WORKERS = NUM_CORES * NUM_SUBCORES
BPW = BATCH // NUM_WORKERS
NBUF = 8
ROUNDS = MAXLEN // NBUF
OUT_RPW = BPW * EMBED_DIM // 128


def _pool_kernel_body(idxT_hbm, table_hbm, pool_hbm, idx_v, out_v, *rest):
    accs = rest[:NBUF]
    sems = rest[NBUF:]
    wid = lax.axis_index("s") * NUM_CORES + lax.axis_index("c")

    pltpu.sync_copy(idxT_hbm.at[:, wid], idx_v)

    for b in range(NBUF):
        pltpu.async_copy(table_hbm.at[idx_v.at[b]], accs[b], sems[b])

    def round_body(r, carry):
        for b in range(NBUF):
            pltpu.make_async_copy(table_hbm.at[idx_v.at[b]], accs[b], sems[b]).wait()
            pltpu.async_copy(
                table_hbm.at[idx_v.at[r * NBUF + b]], accs[b], sems[b], add=True
            )
        return carry

    lax.fori_loop(1, ROUNDS, round_body, 0)
    for b in range(NBUF):
        pltpu.make_async_copy(table_hbm.at[idx_v.at[b]], accs[b], sems[b]).wait()

    def fold_body(orow, carry):
        for j in range(8):
            r = orow * 4 + j // 2
            sl = pl.ds((j % 2) * 16, 16)
            acc = accs[0][r, sl]
            for b in range(1, NBUF):
                acc = acc + accs[b][r, sl]
            out_v[orow, pl.ds(j * 16, 16)] = acc
        return carry

    lax.fori_loop(0, OUT_RPW, fold_body, 0)
    pltpu.sync_copy(out_v, pool_hbm.at[pl.ds(wid * OUT_RPW, OUT_RPW)])


_pool = pl.kernel(
    _pool_kernel_body,
    out_type=jax.ShapeDtypeStruct((BATCH * EMBED_DIM // 128, 128), jnp.float32),
    mesh=plsc.VectorSubcoreMesh(
        core_axis_name="c",
        subcore_axis_name="s",
        num_cores=NUM_CORES,
        num_subcores=NUM_SUBCORES,
    ),
    scratch_types=(
        [
            pltpu.VMEM((MAXLEN, BPW), jnp.int32),
            pltpu.VMEM((OUT_RPW, 128), jnp.float32),
        ]
        + [pltpu.VMEM((BPW, EMBED_DIM), jnp.float32) for _ in range(NBUF)]
        + [pltpu.SemaphoreType.DMA for _ in range(NBUF)]
    ),
    compiler_params=pltpu.CompilerParams(use_tc_tiling_on_sc=False),
)


def _mlp_body(pool_ref, w1_ref, b1_ref, w2_ref, b2_ref, out_ref):
    x = pool_ref[...] * (1.0 / MAXLEN)
    h = jnp.dot(x, w1_ref[...], preferred_element_type=jnp.float32) + b1_ref[...]
    h = jnp.maximum(h, 0.0)
    logits = jnp.dot(h, w2_ref[...], preferred_element_type=jnp.float32) + b2_ref[...]
    m = jnp.max(logits, axis=-1, keepdims=True)
    e = jnp.exp(logits - m)
    out_ref[...] = e / jnp.sum(e, axis=-1, keepdims=True)


_mlp = pl.pallas_call(
    _mlp_body,
    out_shape=jax.ShapeDtypeStruct((BATCH, CLASS_NUM), jnp.float32),
)


def kernel(inputs, table, W1, b1, W2, b2):
    idxT = inputs.T.reshape(MAXLEN, NUM_WORKERS, BPW)
    pool = _pool(idxT, table).reshape(BATCH, EMBED_DIM)
    return _mlp(pool, W1, b1.reshape(1, HIDDEN), W2, b2.reshape(1, CLASS_NUM))
